# Initial kernel scaffold; baseline (speedup 1.0000x reference)
import jax
import jax.numpy as jnp
from jax.experimental import pallas as pl

TRACKS = 32


def kernel(x, W, b):
    scores = x @ W + b
    T = scores.shape[0]
    log_small = jnp.log(jnp.asarray(1e-46, dtype=jnp.float32))
    idx = jnp.argmax(scores)
    mask = jnp.ones((T,), dtype=jnp.float32).at[idx].set(log_small)
    slice_ = scores @ mask[None, :]
    result = jnp.concatenate([scores] + [slice_] * TRACKS, axis=1)
    return result


# speedup vs baseline: 2.4206x; 2.4206x over previous
import jax
import jax.numpy as jnp
from jax.experimental import pallas as pl
from jax.experimental.pallas import tpu as pltpu

_T = 1024
_S = 32
_F = 512
_COLS = 1 + _S * _T


def _body(x_ref, w_ref, b_ref, out_ref, scores_ref, sel_ref):
    j = pl.program_id(0)

    @pl.when(j == 0)
    def _init():
        scores = jnp.dot(x_ref[...], w_ref[...],
                         preferred_element_type=jnp.float32) + b_ref[0, 0]
        scores_ref[...] = scores
        mx = jnp.max(scores)
        rows = jax.lax.broadcasted_iota(jnp.int32, scores.shape, 0)
        idx = jnp.min(jnp.where(scores == mx, rows, jnp.int32(_T)))
        sel_ref[0] = jax.lax.rem(idx + 1, jnp.int32(_T))

    sel = sel_ref[0]
    cols = jax.lax.broadcasted_iota(jnp.int32, (1, _T), 1)
    m = jnp.where(cols == sel, jnp.float32(-jnp.inf), jnp.float32(1.0))
    m = jnp.where((j == 0) & (cols == 0), jnp.float32(1.0), m)
    out_ref[...] = scores_ref[...] * m


def kernel(x, W, b):
    b2 = b.reshape(1, 1)
    grid = (_COLS + _T - 1) // _T
    return pl.pallas_call(
        _body,
        grid=(grid,),
        in_specs=[
            pl.BlockSpec((_T, _F), lambda j: (0, 0)),
            pl.BlockSpec((_F, 1), lambda j: (0, 0)),
            pl.BlockSpec((1, 1), lambda j: (0, 0)),
        ],
        out_specs=pl.BlockSpec((_T, _T), lambda j: (0, j)),
        out_shape=jax.ShapeDtypeStruct((_T, _COLS), jnp.float32),
        scratch_shapes=[
            pltpu.VMEM((_T, 1), jnp.float32),
            pltpu.SMEM((1,), jnp.int32),
        ],
    )(x, W, b2)
